# baseline (speedup 1.0000x reference)
import functools
import math

import numpy as np
import jax
import jax.numpy as jnp
from jax import lax
from jax.experimental import pallas as pl
from jax.experimental.pallas import tpu as pltpu
from jax.experimental.pallas import tpu_sc as plsc

_BATCH, _SEQ = 16, 8192
_N = _BATCH * _SEQ
_TEMPERATURE = 10000.0
_SCALE = 2.0 * math.pi
_R_MAX = 6000.0
_NUM_R_BINS = 50
_NUM_PHI_BINS = 36
_TAB_R = _NUM_R_BINS * 64
_TAB_W = _TAB_R + _NUM_PHI_BINS * 64

_NC, _NS = 2, 16
_NW = _NC * _NS
_BPW = _N // _NW
_CH = 128
_NCH = _BPW // _CH
_NBUF = 4

_RA = 64
_BN_SINE = 512

_ATAN_C = (0.9999798536300659, -0.3326554298400879, 0.1936698853969574,
           -0.11664997786283493, 0.05282219499349594, -0.011769973672926426)
_SIN_C = (6.278553009033203, -41.0910758972168, 77.90902709960938,
          -56.037471771240234)
_RND = 12582912.0


def _sine_consts():
    i = np.arange(64)
    dim_t = _TEMPERATURE ** (2.0 * np.floor(i / 2.0) / 64.0)
    w_half = 1.0 / dim_t
    ph_half = np.where(i % 2 == 1, 0.25, 0.0)
    w = np.concatenate([w_half, w_half]).astype(np.float32)
    ph = np.concatenate([ph_half, ph_half]).astype(np.float32)
    return np.stack([w, ph])


_WP_CONST = _sine_consts()


def _bins_body(x_ref, y_ref, rbi_ref, pbi_ref):
    x = x_ref[...]
    y = y_ref[...]
    s = x * x + y * y
    i = lax.bitcast_convert_type(s, jnp.int32)
    i = 0x5F3759DF - lax.shift_right_logical(i, 1)
    g = lax.bitcast_convert_type(i, jnp.float32)
    hs = 0.5 * s
    g = g * (1.5 - hs * g * g)
    g = g * (1.5 - hs * g * g)
    r = s * g
    rb = jnp.clip((r * (49.0 / _R_MAX)).astype(jnp.int32), 0, 49)

    ax = jnp.abs(x)
    ay = jnp.abs(y)
    hi = jnp.maximum(ax, ay)
    lo = jnp.minimum(ax, ay)
    rc = pl.reciprocal(hi, approx=True)
    rc = rc * (2.0 - hi * rc)
    t = lo * rc
    z = t * t
    a = _ATAN_C[5]
    for k in (4, 3, 2, 1, 0):
        a = a * z + _ATAN_C[k]
    a = a * t
    a = jnp.where(ay > ax, (math.pi / 2) - a, a)
    a = jnp.where(x < 0.0, math.pi - a, a)
    phi = jnp.where(y < 0.0, -a, a)
    pb = ((phi + math.pi) * (35.0 / (2.0 * math.pi))).astype(jnp.int32)
    pb = jnp.clip(pb, 0, 35)
    rbi_ref[...] = rb * 64
    pbi_ref[...] = pb * 64 + _TAB_R


def _sine_body(rad_ref, pos_ref, wp_ref, out_ref):
    p = pos_ref[...]
    x = p[:, 0]
    y = p[:, 1]
    xq = jnp.clip((x + 3000.0) * (1.0 / 6000.0), 0.0, 1.0)
    yq = jnp.clip((y + 2000.0) * (1.0 / 4000.0), 0.0, 1.0)
    w = wp_ref[0]
    ph = wp_ref[1]
    col = lax.broadcasted_iota(jnp.int32, (_BN_SINE, 128), 1)
    th = jnp.where(col < 64, xq[:, None], yq[:, None]) * w[None, :] + ph[None, :]
    u = th - ((th + _RND) - _RND)
    z = u * u
    sv = _SIN_C[3]
    for k in (2, 1, 0):
        sv = sv * z + _SIN_C[k]
    out_ref[:, :128] = sv * u
    out_ref[:, 128:] = rad_ref[...]


@functools.cache
def _make_sc_radial():
    mesh = plsc.VectorSubcoreMesh(core_axis_name="c", subcore_axis_name="s")
    return functools.partial(
        pl.kernel,
        out_type=jax.ShapeDtypeStruct((_N, 128), jnp.float32),
        mesh=mesh,
        compiler_params=pltpu.CompilerParams(needs_layout_passes=False),
        scratch_types=[
            pltpu.VMEM((_TAB_W,), jnp.float32),
            pltpu.VMEM((_NCH, _CH), jnp.int32),
            pltpu.VMEM((_NCH, _CH), jnp.int32),
        ] + [pltpu.VMEM((_CH, 128), jnp.float32) for _ in range(_NBUF)] + [
            pltpu.SemaphoreType.DMA,
        ],
    )(_sc_radial_body)


def _sc_radial_body(tab_hbm, rbi_hbm, pbi_hbm, out_hbm,
                    tab_v, rbi_v, pbi_v, r0, r1, r2, r3, ssem):
    rows_refs = (r0, r1, r2, r3)
    wid = lax.axis_index("s") * _NC + lax.axis_index("c")
    row0 = wid * _BPW
    pltpu.sync_copy(tab_hbm, tab_v)
    pltpu.sync_copy(rbi_hbm.at[pl.ds(wid * _NCH, _NCH)], rbi_v)
    pltpu.sync_copy(pbi_hbm.at[pl.ds(wid * _NCH, _NCH)], pbi_v)

    def _assemble(c, rows_b):
        @pl.loop(0, _CH // 16)
        def _grp(g):
            p0 = g * 16
            rbi = rbi_v[c, pl.ds(p0, 16)]
            pbi = pbi_v[c, pl.ds(p0, 16)]
            pt = lax.iota(jnp.int32, 16) + p0
            for j in range(64):
                jv = jnp.full((16,), j, jnp.int32)
                v = plsc.load_gather(tab_v, [rbi + j])
                plsc.store_scatter(rows_b, [pt, jv], v)
                v2 = plsc.load_gather(tab_v, [pbi + j])
                plsc.store_scatter(rows_b, [pt, jv + 64], v2)

    def _store(c, rows_b):
        pltpu.async_copy(rows_b, out_hbm.at[pl.ds(row0 + c * _CH, _CH)], ssem)

    def _store_drain():
        pltpu.make_async_copy(
            rows_refs[0], out_hbm.at[pl.ds(row0, _CH)], ssem).wait()

    @pl.loop(0, _NCH, step=_NBUF)
    def _chunks(c0):
        for b in range(_NBUF):
            cc = c0 + b

            @pl.when(cc >= _NBUF)
            def _():
                _store_drain()

            _assemble(cc, rows_refs[b])
            _store(cc, rows_refs[b])

    for _ in range(_NBUF):
        _store_drain()


def kernel(positions, r_embed, phi_embed):
    pos2 = positions.reshape(_N, 2)
    xcol = positions[..., 0].reshape(_N // 128, 128)
    ycol = positions[..., 1].reshape(_N // 128, 128)

    rbi2, pbi2 = pl.pallas_call(
        _bins_body,
        grid=(_N // (_RA * 128),),
        in_specs=[
            pl.BlockSpec((_RA, 128), lambda i: (i, 0)),
            pl.BlockSpec((_RA, 128), lambda i: (i, 0)),
        ],
        out_specs=[
            pl.BlockSpec((_RA, 128), lambda i: (i, 0)),
            pl.BlockSpec((_RA, 128), lambda i: (i, 0)),
        ],
        out_shape=[
            jax.ShapeDtypeStruct((_N // 128, 128), jnp.int32),
            jax.ShapeDtypeStruct((_N // 128, 128), jnp.int32),
        ],
    )(xcol, ycol)

    tab = jnp.concatenate([r_embed.reshape(-1), phi_embed.reshape(-1)])
    radial = _make_sc_radial()(tab, rbi2, pbi2)

    out = pl.pallas_call(
        _sine_body,
        grid=(_N // _BN_SINE,),
        in_specs=[
            pl.BlockSpec((_BN_SINE, 128), lambda i: (i, 0)),
            pl.BlockSpec((_BN_SINE, 2), lambda i: (i, 0)),
            pl.BlockSpec((2, 128), lambda i: (0, 0)),
        ],
        out_specs=pl.BlockSpec((_BN_SINE, 256), lambda i: (i, 0)),
        out_shape=jax.ShapeDtypeStruct((_N, 256), jnp.float32),
    )(radial, pos2, jnp.asarray(_WP_CONST))

    return out.reshape(_BATCH, _SEQ, 256)


# speedup vs baseline: 3.6913x; 3.6913x over previous
import functools
import math

import numpy as np
import jax
import jax.numpy as jnp
from jax import lax
from jax.experimental import pallas as pl
from jax.experimental.pallas import tpu as pltpu
from jax.experimental.pallas import tpu_sc as plsc

_BATCH, _SEQ = 16, 8192
_N = _BATCH * _SEQ
_TEMPERATURE = 10000.0
_SCALE = 2.0 * math.pi
_R_MAX = 6000.0
_NUM_R_BINS = 50
_NUM_PHI_BINS = 36
_NUM_FUSED = _NUM_R_BINS * _NUM_PHI_BINS

_NC, _NS = 2, 16
_NW = _NC * _NS
_BPW = _N // _NW
_CH = 128
_NCH = _BPW // _CH
_NBUF = 4
_GAH = 2
_D0 = _NBUF - _GAH

_RA = 64
_BN_SINE = 1024

_ATAN_C = (0.9999798536300659, -0.3326554298400879, 0.1936698853969574,
           -0.11664997786283493, 0.05282219499349594, -0.011769973672926426)
_SIN_C = (6.278553009033203, -41.0910758972168, 77.90902709960938,
          -56.037471771240234)
_RND = 12582912.0


def _sine_consts():
    i = np.arange(64)
    dim_t = _TEMPERATURE ** (2.0 * np.floor(i / 2.0) / 64.0)
    w_half = 1.0 / dim_t
    ph_half = np.where(i % 2 == 1, 0.25, 0.0)
    w = np.concatenate([w_half, w_half]).astype(np.float32)
    ph = np.concatenate([ph_half, ph_half]).astype(np.float32)
    return np.stack([w, ph])


_WP_CONST = _sine_consts()


def _bins_body(x_ref, y_ref, idx_ref):
    x = x_ref[...]
    y = y_ref[...]
    s = x * x + y * y
    i = lax.bitcast_convert_type(s, jnp.int32)
    i = 0x5F3759DF - lax.shift_right_logical(i, 1)
    g = lax.bitcast_convert_type(i, jnp.float32)
    hs = 0.5 * s
    g = g * (1.5 - hs * g * g)
    g = g * (1.5 - hs * g * g)
    r = s * g
    rb = jnp.clip((r * (49.0 / _R_MAX)).astype(jnp.int32), 0, 49)

    ax = jnp.abs(x)
    ay = jnp.abs(y)
    hi = jnp.maximum(ax, ay)
    lo = jnp.minimum(ax, ay)
    rc = pl.reciprocal(hi, approx=True)
    rc = rc * (2.0 - hi * rc)
    t = lo * rc
    z = t * t
    a = _ATAN_C[5]
    for k in (4, 3, 2, 1, 0):
        a = a * z + _ATAN_C[k]
    a = a * t
    a = jnp.where(ay > ax, (math.pi / 2) - a, a)
    a = jnp.where(x < 0.0, math.pi - a, a)
    phi = jnp.where(y < 0.0, -a, a)
    pb = ((phi + math.pi) * (35.0 / (2.0 * math.pi))).astype(jnp.int32)
    pb = jnp.clip(pb, 0, 35)
    idx_ref[...] = rb * _NUM_PHI_BINS + pb


def _sine_body(_, pos_ref, wp_ref, out_ref):
    p = pos_ref[...]
    x = p[:, 0]
    y = p[:, 1]
    xq = jnp.clip((x + 3000.0) * (1.0 / 6000.0), 0.0, 1.0)
    yq = jnp.clip((y + 2000.0) * (1.0 / 4000.0), 0.0, 1.0)
    w = wp_ref[0]
    ph = wp_ref[1]
    col = lax.broadcasted_iota(jnp.int32, (_BN_SINE, 128), 1)
    th = jnp.where(col < 64, xq[:, None], yq[:, None]) * w[None, :] + ph[None, :]
    u = th - ((th + _RND) - _RND)
    z = u * u
    sv = _SIN_C[3]
    for k in (2, 1, 0):
        sv = sv * z + _SIN_C[k]
    out_ref[...] = sv * u


@functools.cache
def _make_sc_gather():
    mesh = plsc.VectorSubcoreMesh(core_axis_name="c", subcore_axis_name="s")
    return functools.partial(
        pl.kernel,
        out_type=jax.ShapeDtypeStruct((_N, 256), jnp.float32),
        mesh=mesh,
        scratch_types=[
            pltpu.VMEM_SHARED((_NUM_FUSED, 128), jnp.float32),
            pltpu.VMEM((_NCH, _CH), jnp.int32),
            pltpu.VMEM((_NBUF, _CH, 128), jnp.float32),
            pltpu.SemaphoreType.DMA,
            pltpu.SemaphoreType.DMA,
        ],
    )(_sc_gather_body)


def _sc_gather_body(table_hbm, idx_hbm, out_hbm, tab_sh, idx_v, rows_v, gsem, ssem):
    sid = lax.axis_index("s")
    wid = sid * _NC + lax.axis_index("c")
    row0 = wid * _BPW

    @pl.when(sid == 0)
    def _():
        pltpu.sync_copy(table_hbm, tab_sh)

    pltpu.sync_copy(idx_hbm.at[pl.ds(wid * _NCH, _NCH)], idx_v)
    plsc.subcore_barrier()

    def _gather(c, b):
        pltpu.async_copy(tab_sh.at[idx_v.at[c]], rows_v.at[b], gsem)

    def _gather_wait(c, b):
        pltpu.make_async_copy(tab_sh.at[idx_v.at[c]], rows_v.at[b], gsem).wait()

    def _store(c, b):
        dst = out_hbm.at[pl.ds(row0 + c * _CH, _CH), pl.ds(128, 128)]
        pltpu.async_copy(rows_v.at[b], dst, ssem)

    def _store_drain():
        dst = out_hbm.at[pl.ds(row0, _CH), pl.ds(128, 128)]
        pltpu.make_async_copy(rows_v.at[0], dst, ssem).wait()

    for j in range(_GAH):
        _gather(j, j)

    @pl.loop(0, _NCH, step=_NBUF)
    def _chunks(c0):
        for b in range(_NBUF):
            cc = c0 + b
            g = cc + _GAH

            @pl.when(cc >= _D0)
            def _():
                _store_drain()

            @pl.when(g < _NCH)
            def _():
                _gather(g, (b + _GAH) % _NBUF)

            _gather_wait(cc, b)
            _store(cc, b)

    for _ in range(_D0):
        _store_drain()


def _fused_table(r_embed, phi_embed):
    return jnp.concatenate(
        [
            jnp.broadcast_to(r_embed[:, None, :], (_NUM_R_BINS, _NUM_PHI_BINS, 64)),
            jnp.broadcast_to(phi_embed[None, :, :], (_NUM_R_BINS, _NUM_PHI_BINS, 64)),
        ],
        axis=-1,
    ).reshape(_NUM_FUSED, 128)


def kernel(positions, r_embed, phi_embed):
    pos2 = positions.reshape(_N, 2)
    xcol = positions[..., 0].reshape(_N // 128, 128)
    ycol = positions[..., 1].reshape(_N // 128, 128)

    idx2 = pl.pallas_call(
        _bins_body,
        grid=(_N // (_RA * 128),),
        in_specs=[
            pl.BlockSpec((_RA, 128), lambda i: (i, 0)),
            pl.BlockSpec((_RA, 128), lambda i: (i, 0)),
        ],
        out_specs=pl.BlockSpec((_RA, 128), lambda i: (i, 0)),
        out_shape=jax.ShapeDtypeStruct((_N // 128, 128), jnp.int32),
    )(xcol, ycol)

    comb = _make_sc_gather()(_fused_table(r_embed, phi_embed), idx2)

    comb = pl.pallas_call(
        _sine_body,
        grid=(_N // _BN_SINE,),
        in_specs=[
            pl.BlockSpec(memory_space=pl.ANY),
            pl.BlockSpec((_BN_SINE, 2), lambda i: (i, 0)),
            pl.BlockSpec((2, 128), lambda i: (0, 0)),
        ],
        out_specs=pl.BlockSpec((_BN_SINE, 128), lambda i: (i, 0)),
        out_shape=jax.ShapeDtypeStruct((_N, 256), jnp.float32),
        input_output_aliases={0: 0},
    )(comb, pos2, jnp.asarray(_WP_CONST))

    return comb.reshape(_BATCH, _SEQ, 256)
